# baseline (speedup 1.0000x reference)
import jax
import jax.numpy as jnp
from jax.experimental import pallas as pl
from jax.experimental.pallas import tpu as pltpu

_R = 2000


def _body(nc_ref, t_ref, x_ref, out_ref):
    t = t_ref[...]
    c0 = nc_ref[0, 0]
    c1 = nc_ref[1, 0]
    c2 = nc_ref[2, 0]
    c3 = nc_ref[3, 0]
    f = jnp.where(t == 0, c0, jnp.where(t == 1, c1, jnp.where(t == 2, c2, c3)))
    out_ref[...] = x_ref[...] * f


def kernel(node_features, atom_types, norm_const):
    n, d = node_features.shape
    t2d = atom_types.astype(jnp.int32).reshape(n, 1)
    out_features = pl.pallas_call(
        _body,
        grid=(n // _R,),
        in_specs=[
            pl.BlockSpec(memory_space=pltpu.SMEM),
            pl.BlockSpec((_R, 1), lambda i: (i, 0)),
            pl.BlockSpec((_R, d), lambda i: (i, 0)),
        ],
        out_specs=pl.BlockSpec((_R, d), lambda i: (i, 0)),
        out_shape=jax.ShapeDtypeStruct((n, d), jnp.float32),
    )(norm_const, t2d, node_features)
    return out_features, jnp.zeros((n, 1), jnp.float32)


# speedup vs baseline: 1.4393x; 1.4393x over previous
import jax
import jax.numpy as jnp
from jax.experimental import pallas as pl
from jax.experimental.pallas import tpu as pltpu

_R = 2000


def _body(nc_ref, t_ref, x_ref, out_ref, nf_ref):
    t = t_ref[0]
    c0 = nc_ref[0, 0]
    c1 = nc_ref[1, 0]
    c2 = nc_ref[2, 0]
    c3 = nc_ref[3, 0]
    f = jnp.where(t == 0, c0, jnp.where(t == 1, c1, jnp.where(t == 2, c2, c3)))
    nf_ref[0] = f
    f_col = jnp.transpose(f, (1, 0))
    out_ref[...] = x_ref[...] * f_col


def kernel(node_features, atom_types, norm_const):
    n, d = node_features.shape
    g = n // _R
    t3d = atom_types.astype(jnp.int32).reshape(g, 1, _R)
    out_features, nf3d = pl.pallas_call(
        _body,
        grid=(g,),
        in_specs=[
            pl.BlockSpec(memory_space=pltpu.SMEM),
            pl.BlockSpec((1, 1, _R), lambda i: (i, 0, 0)),
            pl.BlockSpec((_R, d), lambda i: (i, 0)),
        ],
        out_specs=[
            pl.BlockSpec((_R, d), lambda i: (i, 0)),
            pl.BlockSpec((1, 1, _R), lambda i: (i, 0, 0)),
        ],
        out_shape=[
            jax.ShapeDtypeStruct((n, d), jnp.float32),
            jax.ShapeDtypeStruct((g, 1, _R), jnp.float32),
        ],
    )(norm_const, t3d, node_features)
    return out_features, nf3d.reshape(n, 1)


# speedup vs baseline: 1.6752x; 1.1639x over previous
import jax
import jax.numpy as jnp
from jax.experimental import pallas as pl
from jax.experimental.pallas import tpu as pltpu

_R = 4000


def _body(nc_ref, t_ref, x_ref, out_ref, nf_ref):
    t = t_ref[0]
    c0 = nc_ref[0, 0]
    c1 = nc_ref[1, 0]
    c2 = nc_ref[2, 0]
    c3 = nc_ref[3, 0]
    f = jnp.where(t == 0, c0, jnp.where(t == 1, c1, jnp.where(t == 2, c2, c3)))
    nf_ref[0] = f
    f_col = jnp.transpose(f, (1, 0))
    out_ref[...] = x_ref[...] * f_col


def kernel(node_features, atom_types, norm_const):
    n, d = node_features.shape
    g = n // _R
    t3d = atom_types.astype(jnp.int32).reshape(g, 1, _R)
    out_features, nf3d = pl.pallas_call(
        _body,
        grid=(g,),
        in_specs=[
            pl.BlockSpec(memory_space=pltpu.SMEM),
            pl.BlockSpec((1, 1, _R), lambda i: (i, 0, 0)),
            pl.BlockSpec((_R, d), lambda i: (i, 0)),
        ],
        out_specs=[
            pl.BlockSpec((_R, d), lambda i: (i, 0)),
            pl.BlockSpec((1, 1, _R), lambda i: (i, 0, 0)),
        ],
        out_shape=[
            jax.ShapeDtypeStruct((n, d), jnp.float32),
            jax.ShapeDtypeStruct((g, 1, _R), jnp.float32),
        ],
        compiler_params=pltpu.CompilerParams(
            dimension_semantics=("arbitrary",),
        ),
    )(norm_const, t3d, node_features)
    return out_features, nf3d.reshape(n, 1)


# speedup vs baseline: 1.7060x; 1.0184x over previous
import jax
import jax.numpy as jnp
from jax.experimental import pallas as pl
from jax.experimental.pallas import tpu as pltpu

_R = 5000


def _body(nc_ref, t_ref, x_ref, out_ref, nf_ref):
    t = t_ref[0]
    c0 = nc_ref[0, 0]
    c1 = nc_ref[1, 0]
    c2 = nc_ref[2, 0]
    c3 = nc_ref[3, 0]
    f = jnp.where(t == 0, c0, jnp.where(t == 1, c1, jnp.where(t == 2, c2, c3)))
    nf_ref[0] = f
    f_col = jnp.transpose(f, (1, 0))
    out_ref[...] = x_ref[...] * f_col


def kernel(node_features, atom_types, norm_const):
    n, d = node_features.shape
    g = n // _R
    t3d = atom_types.astype(jnp.int32).reshape(g, 1, _R)
    out_features, nf3d = pl.pallas_call(
        _body,
        grid=(g,),
        in_specs=[
            pl.BlockSpec(memory_space=pltpu.SMEM),
            pl.BlockSpec((1, 1, _R), lambda i: (i, 0, 0)),
            pl.BlockSpec((_R, d), lambda i: (i, 0)),
        ],
        out_specs=[
            pl.BlockSpec((_R, d), lambda i: (i, 0)),
            pl.BlockSpec((1, 1, _R), lambda i: (i, 0, 0)),
        ],
        out_shape=[
            jax.ShapeDtypeStruct((n, d), jnp.float32),
            jax.ShapeDtypeStruct((g, 1, _R), jnp.float32),
        ],
        compiler_params=pltpu.CompilerParams(
            dimension_semantics=("arbitrary",),
        ),
    )(norm_const, t3d, node_features)
    return out_features, nf3d.reshape(n, 1)


# speedup vs baseline: 1.7327x; 1.0156x over previous
import jax
import jax.numpy as jnp
from jax.experimental import pallas as pl
from jax.experimental.pallas import tpu as pltpu

_R = 10000


def _body(nc_ref, t_ref, x_ref, out_ref, nf_ref):
    t = t_ref[0]
    c0 = nc_ref[0, 0]
    c1 = nc_ref[1, 0]
    c2 = nc_ref[2, 0]
    c3 = nc_ref[3, 0]
    f = jnp.where(t == 0, c0, jnp.where(t == 1, c1, jnp.where(t == 2, c2, c3)))
    nf_ref[0] = f
    f_col = jnp.transpose(f, (1, 0))
    out_ref[...] = x_ref[...] * f_col


def kernel(node_features, atom_types, norm_const):
    n, d = node_features.shape
    g = n // _R
    t3d = atom_types.astype(jnp.int32).reshape(g, 1, _R)
    out_features, nf3d = pl.pallas_call(
        _body,
        grid=(g,),
        in_specs=[
            pl.BlockSpec(memory_space=pltpu.SMEM),
            pl.BlockSpec((1, 1, _R), lambda i: (i, 0, 0)),
            pl.BlockSpec((_R, d), lambda i: (i, 0)),
        ],
        out_specs=[
            pl.BlockSpec((_R, d), lambda i: (i, 0)),
            pl.BlockSpec((1, 1, _R), lambda i: (i, 0, 0)),
        ],
        out_shape=[
            jax.ShapeDtypeStruct((n, d), jnp.float32),
            jax.ShapeDtypeStruct((g, 1, _R), jnp.float32),
        ],
        compiler_params=pltpu.CompilerParams(
            dimension_semantics=("arbitrary",),
        ),
    )(norm_const, t3d, node_features)
    return out_features, nf3d.reshape(n, 1)
